# Initial kernel scaffold; baseline (speedup 1.0000x reference)
import jax
import jax.numpy as jnp
from jax.experimental import pallas as pl


def kernel(logits, actions):
    raise NotImplementedError("write your pallas kernel here")


# speedup vs baseline: 1.6832x; 1.6832x over previous
import functools

import jax
import jax.numpy as jnp
from jax.experimental import pallas as pl
from jax.experimental.pallas import tpu as pltpu

_B = 32
_V = 1000000
_BV = 16384


def _body(actions_ref, logits_ref, logp_ref, ent_ref, det_ref,
          m_ref, s_ref, t_ref, bi_ref, la_ref):
    i = pl.program_id(0)
    nb = pl.num_programs(0)
    blk = logits_ref[...]
    col0 = i * _BV
    cols = jax.lax.broadcasted_iota(jnp.int32, (_B, _BV), 1) + col0
    valid = cols < _V
    neg_inf = jnp.float32(-jnp.inf)
    blkm = jnp.where(valid, blk, neg_inf)

    bm = jnp.max(blkm, axis=1, keepdims=True)
    barg = jnp.argmax(blkm, axis=1).astype(jnp.int32)[:, None] + col0

    a = actions_ref[...]
    la_c = jnp.sum(jnp.where(cols == a, blkm, 0.0), axis=1, keepdims=True)

    @pl.when(i == 0)
    def _init():
        m_ref[...] = jnp.full((_B, 1), neg_inf, jnp.float32)
        s_ref[...] = jnp.zeros((_B, 1), jnp.float32)
        t_ref[...] = jnp.zeros((_B, 1), jnp.float32)
        bi_ref[...] = jnp.zeros((_B, 1), jnp.int32)
        la_ref[...] = jnp.zeros((_B, 1), jnp.float32)

    m_old = m_ref[...]
    m_new = jnp.maximum(m_old, bm)
    scale = jnp.where(m_old == neg_inf, 0.0, jnp.exp(m_old - m_new))
    e = jnp.exp(blkm - m_new)
    s_new = s_ref[...] * scale + jnp.sum(e, axis=1, keepdims=True)
    t_new = t_ref[...] * scale + jnp.sum(
        e * jnp.where(valid, blk, 0.0), axis=1, keepdims=True)
    bi_new = jnp.where(bm > m_old, barg, bi_ref[...])
    la_new = la_ref[...] + la_c

    m_ref[...] = m_new
    s_ref[...] = s_new
    t_ref[...] = t_new
    bi_ref[...] = bi_new
    la_ref[...] = la_new

    @pl.when(i == nb - 1)
    def _fin():
        log_s = jnp.log(s_new)
        logp_ref[...] = la_new - m_new - log_s
        ent_ref[...] = m_new + log_s - t_new / s_new
        det_ref[...] = bi_new


@functools.partial(jax.jit, static_argnames=("interpret",))
def _run(logits, actions_i32, interpret=False):
    nb = (_V + _BV - 1) // _BV
    grid = (nb,)
    out_shapes = (
        jax.ShapeDtypeStruct((_B, 1), jnp.float32),
        jax.ShapeDtypeStruct((_B, 1), jnp.float32),
        jax.ShapeDtypeStruct((_B, 1), jnp.int32),
    )
    small = pl.BlockSpec((_B, 1), lambda i: (0, 0))
    return pl.pallas_call(
        _body,
        grid=grid,
        in_specs=[
            small,
            pl.BlockSpec((_B, _BV), lambda i: (0, i)),
        ],
        out_specs=(small, small, small),
        out_shape=out_shapes,
        scratch_shapes=[
            pltpu.VMEM((_B, 1), jnp.float32),
            pltpu.VMEM((_B, 1), jnp.float32),
            pltpu.VMEM((_B, 1), jnp.float32),
            pltpu.VMEM((_B, 1), jnp.int32),
            pltpu.VMEM((_B, 1), jnp.float32),
        ],
        interpret=interpret,
    )(actions_i32, logits)


def kernel(logits, actions):
    actions_i32 = actions.astype(jnp.int32)
    log_prob, entropy, deterministic = _run(logits, actions_i32)
    return log_prob, entropy, deterministic
